# baseline (speedup 1.0000x reference)
import functools

import jax
import jax.numpy as jnp
from jax import lax
from jax.experimental import pallas as pl
from jax.experimental.pallas import tpu as pltpu
from jax.experimental.pallas import tpu_sc as plsc

DIM = 64

NC = 2
NS = 16
NW = NC * NS

LANES = 16
BLK = 128
NBUF = 2


def _embed(x, tb2):
    S0, S1 = x.shape
    assert S0 == NW * BLK and S1 % 2 == 0

    mesh = plsc.VectorSubcoreMesh(
        core_axis_name="c", subcore_axis_name="s", num_cores=NC,
        num_subcores=NS)

    @functools.partial(
        pl.kernel,
        out_type=jax.ShapeDtypeStruct((S1, 8, S0 // BLK, 8, BLK), jnp.float32),
        mesh=mesh,
        compiler_params=pltpu.CompilerParams(
            needs_layout_passes=False, disable_bounds_checks=True),
        scratch_types=[
            pltpu.VMEM((S1, BLK), jnp.int32),
            pltpu.VMEM((S1, BLK), jnp.int32),
            pltpu.SemaphoreType.DMA,
            pltpu.SemaphoreType.DMA,
            pltpu.SemaphoreType.DMA,
            pltpu.SemaphoreType.DMA,
        ],
    )
    def k(x_hbm, tb2_hbm, out_hbm, pidx, hoff, g0, g1, o0, o1):
        gsems = (g0, g1)
        osems = (o0, o1)
        wid = lax.axis_index("s") * NC + lax.axis_index("c")
        at = wid

        def prologue(idx_v):
            pltpu.sync_copy(
                x_hbm.at[pl.ds(pl.multiple_of(at * BLK, BLK), BLK)], idx_v)

            @pl.loop(0, S1)
            def _(s):
                s_vec = jnp.full((LANES,), s, jnp.int32)
                for g in range(BLK // LANES):
                    t_vec = lax.iota(jnp.int32, LANES) + g * LANES
                    v = plsc.load_gather(idx_v, [t_vec, s_vec])
                    pidx[s, pl.ds(g * LANES, LANES)] = v >> 1
                    hoff[s, pl.ds(g * LANES, LANES)] = (v & 1) * 64

        pl.run_scoped(prologue, pltpu.VMEM((BLK, S1), jnp.int32))

        def pipeline(prow, st):
            def fire_gather(s, b):
                pltpu.async_copy(tb2_hbm.at[pidx.at[s]], prow.at[b], gsems[b])

            def wait_gather(b):
                pltpu.make_async_copy(
                    tb2_hbm.at[pl.ds(0, BLK)], prow.at[b], gsems[b]).wait()

            def repack(s, b):
                @plsc.parallel_loop(0, BLK // LANES, unroll=4)
                def _(g):
                    off = pl.multiple_of(g * LANES, LANES)
                    iot = lax.iota(jnp.int32, LANES)
                    t_vec = iot + g * LANES
                    h_vec = hoff[s, pl.ds(off, LANES)]
                    for d in range(DIM):
                        d_vec = (iot + d) & (DIM - 1)
                        col = h_vec + d_vec
                        val = plsc.load_gather(prow.at[b], [t_vec, col])
                        plsc.store_scatter(
                            st.at[b], [d_vec >> 3, d_vec & 7, t_vec], val)

            def fire_out(s, b):
                pltpu.async_copy(st.at[b], out_hbm.at[s, :, at], osems[b])

            def wait_out(b):
                pltpu.make_async_copy(
                    st.at[b], out_hbm.at[0, :, 0], osems[b]).wait()

            fire_gather(0, 0)
            fire_gather(1, 1)
            wait_gather(0)
            repack(0, 0)
            fire_out(0, 0)

            @pl.loop(NBUF, S1, step=NBUF)
            def _(i0):
                for d in range(NBUF):
                    i = i0 + d
                    b = d
                    ob = 1 - b
                    wait_out(b)
                    fire_gather(i, b)
                    wait_gather(ob)
                    repack(i - 1, ob)
                    fire_out(i - 1, ob)

            wait_gather(1)
            repack(S1 - 1, 1)
            fire_out(S1 - 1, 1)
            wait_out(0)
            wait_out(1)

        pl.run_scoped(
            pipeline,
            pltpu.VMEM((NBUF, BLK, 128), jnp.float32),
            pltpu.VMEM((NBUF, 8, 8, BLK), jnp.float32))

    return k(x, tb2)


def kernel(x, table):
    s0, s1 = x.shape
    v = table.shape[0]
    tb2 = table.reshape(v // 2, 128)
    r5 = _embed(x.astype(jnp.int32), tb2)
    return r5.transpose((2, 4, 0, 1, 3)).reshape(s0, s1, DIM)


# speedup vs baseline: 1.0405x; 1.0405x over previous
import functools

import jax
import jax.numpy as jnp
from jax import lax
from jax.experimental import pallas as pl
from jax.experimental.pallas import tpu as pltpu
from jax.experimental.pallas import tpu_sc as plsc

DIM = 64

NC = 2
NS = 16
NW = NC * NS

LANES = 16
BLK = 128
NBUF = 2


def _embed(x, tb2):
    S0, S1 = x.shape
    assert S0 == NW * BLK and S1 % 2 == 0

    mesh = plsc.VectorSubcoreMesh(
        core_axis_name="c", subcore_axis_name="s", num_cores=NC,
        num_subcores=NS)

    @functools.partial(
        pl.kernel,
        out_type=jax.ShapeDtypeStruct((S1, 8, S0 // BLK, 8, BLK), jnp.float32),
        mesh=mesh,
        compiler_params=pltpu.CompilerParams(
            needs_layout_passes=False, disable_bounds_checks=True),
        scratch_types=[
            pltpu.VMEM((S1, BLK), jnp.int32),
            pltpu.VMEM((S1, BLK), jnp.int32),
            pltpu.SemaphoreType.DMA,
            pltpu.SemaphoreType.DMA,
            pltpu.SemaphoreType.DMA,
            pltpu.SemaphoreType.DMA,
        ],
    )
    def k(x_hbm, tb2_hbm, out_hbm, pidx, hoff, g0, g1, o0, o1):
        gsems = (g0, g1)
        osems = (o0, o1)
        wid = lax.axis_index("s") * NC + lax.axis_index("c")
        at = wid

        def prologue(idx_v):
            pltpu.sync_copy(
                x_hbm.at[pl.ds(pl.multiple_of(at * BLK, BLK), BLK)], idx_v)

            @pl.loop(0, S1)
            def _(s):
                s_vec = jnp.full((LANES,), s, jnp.int32)
                for g in range(BLK // LANES):
                    t_vec = lax.iota(jnp.int32, LANES) + g * LANES
                    v = plsc.load_gather(idx_v, [t_vec, s_vec])
                    pidx[s, pl.ds(g * LANES, LANES)] = v >> 1
                    hoff[s, pl.ds(g * LANES, LANES)] = (v & 1) * 64

        pl.run_scoped(prologue, pltpu.VMEM((BLK, S1), jnp.int32))

        def pipeline(prow, st):
            def fire_gather(s, b):
                pltpu.async_copy(tb2_hbm.at[pidx.at[s]], prow.at[b], gsems[b])

            def wait_gather(b):
                pltpu.make_async_copy(
                    tb2_hbm.at[pl.ds(0, BLK)], prow.at[b], gsems[b]).wait()

            def repack(s, b):
                @plsc.parallel_loop(0, BLK // LANES, unroll=2)
                def _(g):
                    off = pl.multiple_of(g * LANES, LANES)
                    iot = lax.iota(jnp.int32, LANES)
                    t_vec = iot + g * LANES
                    h_vec = hoff[s, pl.ds(off, LANES)]
                    for d in range(DIM):
                        d_vec = (iot + d) & (DIM - 1)
                        col = h_vec + d_vec
                        val = plsc.load_gather(prow.at[b], [t_vec, col])
                        plsc.store_scatter(
                            st.at[b], [d_vec >> 3, d_vec & 7, t_vec], val)

            def fire_out(s, b):
                pltpu.async_copy(st.at[b], out_hbm.at[s, :, at], osems[b])

            def wait_out(b):
                pltpu.make_async_copy(
                    st.at[b], out_hbm.at[0, :, 0], osems[b]).wait()

            fire_gather(0, 0)
            fire_gather(1, 1)
            wait_gather(0)
            repack(0, 0)
            fire_out(0, 0)

            @pl.loop(NBUF, S1, step=NBUF)
            def _(i0):
                for d in range(NBUF):
                    i = i0 + d
                    b = d
                    ob = 1 - b
                    wait_out(b)
                    fire_gather(i, b)
                    wait_gather(ob)
                    repack(i - 1, ob)
                    fire_out(i - 1, ob)

            wait_gather(1)
            repack(S1 - 1, 1)
            fire_out(S1 - 1, 1)
            wait_out(0)
            wait_out(1)

        pl.run_scoped(
            pipeline,
            pltpu.VMEM((NBUF, BLK, 128), jnp.float32),
            pltpu.VMEM((NBUF, 8, 8, BLK), jnp.float32))

    return k(x, tb2)


def kernel(x, table):
    s0, s1 = x.shape
    v = table.shape[0]
    tb2 = table.reshape(v // 2, 128)
    r5 = _embed(x.astype(jnp.int32), tb2)
    return r5.transpose((2, 4, 0, 1, 3)).reshape(s0, s1, DIM)
